# baseline (speedup 1.0000x reference)
import functools

import jax
import jax.numpy as jnp
from jax import lax
from jax.experimental import pallas as pl
from jax.experimental.pallas import tpu as pltpu
from jax.experimental.pallas import tpu_sc as plsc

_NUM_CORES = 2
_NUM_SUBCORES = 16
_NW = _NUM_CORES * _NUM_SUBCORES
_LANES = 16
_CHUNK = 4
_NBUF = 4
_AHEAD = 2

_SEQ_SC = 2048
_TC_BLK = 256


def _make_sc_kernel(seq_sc, batch, d_model):
    assert seq_sc % (_NW * _CHUNK) == 0
    per_w = seq_sc // _NW
    n_chunks = per_w // _CHUNK
    mesh = plsc.VectorSubcoreMesh(
        core_axis_name="c", subcore_axis_name="s")

    @functools.partial(
        pl.kernel,
        out_type=jax.ShapeDtypeStruct((seq_sc, batch, d_model), jnp.float32),
        mesh=mesh,
        scratch_types=[
            pltpu.VMEM((n_chunks, _CHUNK), jnp.int32),
            pltpu.VMEM((_NBUF, _CHUNK, d_model), jnp.float32),
            pltpu.VMEM((_NBUF, _CHUNK, batch, d_model), jnp.float32),
            pltpu.SemaphoreType.DMA((_NBUF,)),
            pltpu.SemaphoreType.DMA((_NBUF,)),
        ],
    )
    def _k(x_hbm, pos_hbm, ids_hbm, out_hbm, idx_v, pos_v, x_v, lsem, ssem):
        wid = lax.axis_index("s") * _NUM_CORES + lax.axis_index("c")
        base0 = wid * per_w

        def issue_loads(c, r):
            base = base0 + c * _CHUNK
            pltpu.async_copy(pos_hbm.at[idx_v.at[c]], pos_v.at[r],
                             lsem.at[r])
            pltpu.async_copy(x_hbm.at[pl.ds(base, _CHUNK)], x_v.at[r],
                             lsem.at[r])

        def wait_loads(c, r):
            base = base0 + c * _CHUNK
            pltpu.make_async_copy(pos_hbm.at[idx_v.at[c]], pos_v.at[r],
                                  lsem.at[r]).wait()
            pltpu.make_async_copy(x_hbm.at[pl.ds(base, _CHUNK)], x_v.at[r],
                                  lsem.at[r]).wait()

        def wait_store(c, r):
            base = base0 + c * _CHUNK
            pltpu.make_async_copy(x_v.at[r], out_hbm.at[pl.ds(base, _CHUNK)],
                                  ssem.at[r]).wait()

        pltpu.sync_copy(ids_hbm.at[pl.ds(wid * n_chunks, n_chunks)], idx_v)
        for c in range(_AHEAD):
            issue_loads(c, c % _NBUF)

        def body(i, carry):
            r = lax.rem(i, _NBUF)
            nxt = i + _AHEAD
            rp = lax.rem(nxt, _NBUF)

            @pl.when(nxt < n_chunks)
            def _():
                @pl.when(nxt >= _NBUF)
                def _():
                    wait_store(nxt - _NBUF, rp)
                issue_loads(nxt, rp)

            wait_loads(i, r)
            grp = 8
            for p in range(_CHUNK):
                for j0 in range(0, d_model // _LANES, grp):
                    pvs = [pos_v[r, p, pl.ds((j0 + g) * _LANES, _LANES)]
                           for g in range(grp)]
                    for g in range(grp):
                        sl = pl.ds((j0 + g) * _LANES, _LANES)
                        for b in range(batch):
                            plsc.addupdate(x_v.at[r, p, b, sl], pvs[g])
            base = base0 + i * _CHUNK
            pltpu.async_copy(x_v.at[r], out_hbm.at[pl.ds(base, _CHUNK)],
                             ssem.at[r])
            return carry

        lax.fori_loop(0, n_chunks, body, 0)
        for k in range(_NBUF):
            wait_store(n_chunks - _NBUF + k,
                       (n_chunks - _NBUF + k) % _NBUF)

    return _k


def _tc_body(x_ref, pos_ref, o_ref):
    o_ref[...] = x_ref[...] + pos_ref[...][:, None, :]


def _make_tc_kernel(seq_off, seq_tc, batch, d_model):
    assert seq_tc % _TC_BLK == 0 and seq_off % _TC_BLK == 0
    off_blk = seq_off // _TC_BLK
    return pl.pallas_call(
        _tc_body,
        grid=(seq_tc // _TC_BLK,),
        in_specs=[
            pl.BlockSpec((_TC_BLK, batch, d_model),
                         lambda i: (off_blk + i, 0, 0)),
            pl.BlockSpec((_TC_BLK, d_model),
                         lambda i: (off_blk + i, 0)),
        ],
        out_specs=pl.BlockSpec((_TC_BLK, batch, d_model),
                               lambda i: (i, 0, 0)),
        out_shape=jax.ShapeDtypeStruct((seq_tc, batch, d_model),
                                       jnp.float32),
    )


@jax.jit
def kernel(x, pos_table, position_ids):
    seq, batch, d_model = x.shape
    ids = position_ids.reshape(-1, _CHUNK)
    sc_k = _make_sc_kernel(_SEQ_SC, batch, d_model)
    tc_k = _make_tc_kernel(_SEQ_SC, seq - _SEQ_SC, batch, d_model)
    out_sc = sc_k(x, pos_table, ids)
    out_tc = tc_k(x, pos_table)
    return jnp.concatenate([out_sc, out_tc], axis=0)


# speedup vs baseline: 1.9315x; 1.9315x over previous
import functools

import jax
import jax.numpy as jnp
from jax import lax
from jax.experimental import pallas as pl
from jax.experimental.pallas import tpu as pltpu
from jax.experimental.pallas import tpu_sc as plsc

_NUM_CORES = 2
_NUM_SUBCORES = 16
_NW = _NUM_CORES * _NUM_SUBCORES
_LANES = 16
_CHUNK = 4
_NBUF = 5
_AHEAD = 2


def _make_kernel(seq, batch, d_model):
    assert seq % (_NW * _CHUNK) == 0
    per_w = seq // _NW
    n_chunks = per_w // _CHUNK
    mesh = plsc.VectorSubcoreMesh(
        core_axis_name="c", subcore_axis_name="s")

    @functools.partial(
        pl.kernel,
        out_type=jax.ShapeDtypeStruct((seq, batch, d_model), jnp.float32),
        mesh=mesh,
        scratch_types=[
            pltpu.VMEM((n_chunks, _CHUNK), jnp.int32),
            pltpu.VMEM((_NBUF, _CHUNK, d_model), jnp.float32),
            pltpu.VMEM((_NBUF, _CHUNK, batch, d_model), jnp.float32),
            pltpu.SemaphoreType.DMA((_NBUF,)),
            pltpu.SemaphoreType.DMA((_NBUF,)),
        ],
    )
    def _k(x_hbm, pos_hbm, ids_hbm, out_hbm, idx_v, pos_v, x_v, lsem, ssem):
        wid = lax.axis_index("s") * _NUM_CORES + lax.axis_index("c")
        base0 = wid * per_w

        def issue_loads(c, r):
            base = base0 + c * _CHUNK
            pltpu.async_copy(pos_hbm.at[idx_v.at[c]], pos_v.at[r],
                             lsem.at[r])
            pltpu.async_copy(x_hbm.at[pl.ds(base, _CHUNK)], x_v.at[r],
                             lsem.at[r])

        def wait_loads(c, r):
            base = base0 + c * _CHUNK
            pltpu.make_async_copy(pos_hbm.at[idx_v.at[c]], pos_v.at[r],
                                  lsem.at[r]).wait()
            pltpu.make_async_copy(x_hbm.at[pl.ds(base, _CHUNK)], x_v.at[r],
                                  lsem.at[r]).wait()

        def wait_store(c, r):
            base = base0 + c * _CHUNK
            pltpu.make_async_copy(x_v.at[r], out_hbm.at[pl.ds(base, _CHUNK)],
                                  ssem.at[r]).wait()

        pltpu.sync_copy(ids_hbm.at[pl.ds(wid * n_chunks, n_chunks)], idx_v)
        for c in range(_AHEAD):
            issue_loads(c, c % _NBUF)

        def body(i, carry):
            r = lax.rem(i, _NBUF)
            nxt = i + _AHEAD
            rp = lax.rem(nxt, _NBUF)

            @pl.when(nxt < n_chunks)
            def _():
                @pl.when(nxt >= _NBUF)
                def _():
                    wait_store(nxt - _NBUF, rp)
                issue_loads(nxt, rp)

            wait_loads(i, r)
            grp = 8
            for p in range(_CHUNK):
                for j0 in range(0, d_model // _LANES, grp):
                    pvs = [pos_v[r, p, pl.ds((j0 + g) * _LANES, _LANES)]
                           for g in range(grp)]
                    for g in range(grp):
                        sl = pl.ds((j0 + g) * _LANES, _LANES)
                        for b in range(batch):
                            plsc.addupdate(x_v.at[r, p, b, sl], pvs[g])
            base = base0 + i * _CHUNK
            pltpu.async_copy(x_v.at[r], out_hbm.at[pl.ds(base, _CHUNK)],
                             ssem.at[r])
            return carry

        lax.fori_loop(0, n_chunks, body, 0)
        for k in range(_NBUF):
            wait_store(n_chunks - _NBUF + k,
                       (n_chunks - _NBUF + k) % _NBUF)

    return _k


@jax.jit
def kernel(x, pos_table, position_ids):
    seq, batch, d_model = x.shape
    ids = position_ids.reshape(-1, _CHUNK)
    k = _make_kernel(seq, batch, d_model)
    return k(x, pos_table, ids)


# speedup vs baseline: 1.9373x; 1.0030x over previous
import functools

import jax
import jax.numpy as jnp
from jax import lax
from jax.experimental import pallas as pl
from jax.experimental.pallas import tpu as pltpu
from jax.experimental.pallas import tpu_sc as plsc

_NUM_CORES = 2
_NUM_SUBCORES = 16
_NW = _NUM_CORES * _NUM_SUBCORES
_LANES = 16
_CHUNK = 2
_NBUF = 8
_AHEAD = 4


def _make_kernel(seq, batch, d_model):
    assert seq % (_NW * _CHUNK) == 0
    per_w = seq // _NW
    n_chunks = per_w // _CHUNK
    mesh = plsc.VectorSubcoreMesh(
        core_axis_name="c", subcore_axis_name="s")

    @functools.partial(
        pl.kernel,
        out_type=jax.ShapeDtypeStruct((seq, batch, d_model), jnp.float32),
        mesh=mesh,
        scratch_types=[
            pltpu.VMEM((n_chunks, _CHUNK), jnp.int32),
            pltpu.VMEM((_NBUF, _CHUNK, d_model), jnp.float32),
            pltpu.VMEM((_NBUF, _CHUNK, batch, d_model), jnp.float32),
            pltpu.SemaphoreType.DMA((_NBUF,)),
            pltpu.SemaphoreType.DMA((_NBUF,)),
        ],
    )
    def _k(x_hbm, pos_hbm, ids_hbm, out_hbm, idx_v, pos_v, x_v, lsem, ssem):
        wid = lax.axis_index("s") * _NUM_CORES + lax.axis_index("c")
        base0 = wid * per_w

        def issue_loads(c, r):
            base = base0 + c * _CHUNK
            pltpu.async_copy(pos_hbm.at[idx_v.at[c]], pos_v.at[r],
                             lsem.at[r])
            pltpu.async_copy(x_hbm.at[pl.ds(base, _CHUNK)], x_v.at[r],
                             lsem.at[r])

        def wait_loads(c, r):
            base = base0 + c * _CHUNK
            pltpu.make_async_copy(pos_hbm.at[idx_v.at[c]], pos_v.at[r],
                                  lsem.at[r]).wait()
            pltpu.make_async_copy(x_hbm.at[pl.ds(base, _CHUNK)], x_v.at[r],
                                  lsem.at[r]).wait()

        def wait_store(c, r):
            base = base0 + c * _CHUNK
            pltpu.make_async_copy(x_v.at[r], out_hbm.at[pl.ds(base, _CHUNK)],
                                  ssem.at[r]).wait()

        pltpu.sync_copy(ids_hbm.at[pl.ds(wid * n_chunks, n_chunks)], idx_v)
        for c in range(_AHEAD):
            issue_loads(c, c % _NBUF)

        def body(i, carry):
            r = lax.rem(i, _NBUF)
            nxt = i + _AHEAD
            rp = lax.rem(nxt, _NBUF)

            @pl.when(nxt < n_chunks)
            def _():
                @pl.when(nxt >= _NBUF)
                def _():
                    wait_store(nxt - _NBUF, rp)
                issue_loads(nxt, rp)

            wait_loads(i, r)
            grp = 8
            for p in range(_CHUNK):
                for j0 in range(0, d_model // _LANES, grp):
                    pvs = [pos_v[r, p, pl.ds((j0 + g) * _LANES, _LANES)]
                           for g in range(grp)]
                    for g in range(grp):
                        sl = pl.ds((j0 + g) * _LANES, _LANES)
                        for b in range(batch):
                            plsc.addupdate(x_v.at[r, p, b, sl], pvs[g])
            base = base0 + i * _CHUNK
            pltpu.async_copy(x_v.at[r], out_hbm.at[pl.ds(base, _CHUNK)],
                             ssem.at[r])
            return carry

        lax.fori_loop(0, n_chunks, body, 0)
        for k in range(_NBUF):
            wait_store(n_chunks - _NBUF + k,
                       (n_chunks - _NBUF + k) % _NBUF)

    return _k


@jax.jit
def kernel(x, pos_table, position_ids):
    seq, batch, d_model = x.shape
    ids = position_ids.reshape(-1, _CHUNK)
    k = _make_kernel(seq, batch, d_model)
    return k(x, pos_table, ids)
